# baseline (speedup 1.0000x reference)
import functools

import jax
import jax.numpy as jnp
from jax import lax
from jax.experimental import pallas as pl
from jax.experimental.pallas import tpu as pltpu
from jax.experimental.pallas import tpu_sc as plsc

_K = 4
_D = 100000
_SEG = 2000
_NSEG = _D // _SEG
_DC = 20000
_NDC = _D // _DC
_SEG_PER_DC = _DC // _SEG
_VPS = _SEG // 16
_UNROLL = 5
_BIG = 2 ** 30


def _topk_body(logits_hbm, gn_hbm, out_hbm, parr, lbuf, obuf, cm_smem, sel_smem):
    c_id = lax.axis_index("c")
    s_id = lax.axis_index("s")
    wid = s_id * 2 + c_id
    lrow = lax.rem(wid, 16)
    gbase = pl.multiple_of(wid * _D, 8)
    lbase = pl.multiple_of(lrow * _D, 8)

    for dc in range(_NDC):
        pltpu.sync_copy(gn_hbm.at[pl.ds(gbase + dc * _DC, _DC)],
                        parr.at[pl.ds(dc * _DC, _DC)])
        pltpu.sync_copy(logits_hbm.at[pl.ds(lbase + dc * _DC, _DC)], lbuf)

        def seg_body(seg, _, dc=dc):
            def vec_body(v, m, seg=seg, dc=dc):
                for u in range(_UNROLL):
                    off = seg * _SEG + (v * _UNROLL + u) * 16
                    p = parr[pl.ds(dc * _DC + off, 16)] + lbuf[pl.ds(off, 16)]
                    parr[pl.ds(dc * _DC + off, 16)] = p
                    m = jnp.maximum(m, p)
                return m
            m = lax.fori_loop(0, _VPS // _UNROLL, vec_body,
                              jnp.full((16,), -jnp.inf, dtype=jnp.float32))
            cm_smem[dc * _SEG_PER_DC + seg] = jnp.max(m)
            return 0
        lax.fori_loop(0, _SEG_PER_DC, seg_body, 0)

    for k in range(_K):
        sel_smem[k] = jnp.int32(-1)

    def rescan(seg_id):
        s0 = sel_smem[0]
        s1 = sel_smem[1]
        s2 = sel_smem[2]
        s3 = sel_smem[3]
        lane = lax.iota(jnp.int32, 16)

        def vec_body(v, carry):
            mv, mi = carry
            base = seg_id * _SEG + v * 16
            p = parr[pl.ds(base, 16)]
            iv = base + lane
            excl = (iv == s0) | (iv == s1) | (iv == s2) | (iv == s3)
            p = jnp.where(excl, -jnp.inf, p)
            upd = p > mv
            return jnp.where(upd, p, mv), jnp.where(upd, iv, mi)

        mv, mi = lax.fori_loop(
            0, _VPS, vec_body,
            (jnp.full((16,), -jnp.inf, dtype=jnp.float32),
             jnp.full((16,), _BIG, dtype=jnp.int32)))
        m = jnp.max(mv)
        i = jnp.min(jnp.where(mv == m, mi, _BIG))
        return m, i

    for j in range(_K):
        def best_body(c, carry):
            bv, bc = carry
            v = cm_smem[c]
            better = v > bv
            return jnp.where(better, v, bv), jnp.where(better, c, bc)
        _, bc = lax.fori_loop(0, _NSEG, best_body,
                              (jnp.float32(-jnp.inf), jnp.int32(0)))
        _, i = rescan(bc)
        sel_smem[j] = i
        m2, _ = rescan(bc)
        cm_smem[bc] = m2

    lane = lax.iota(jnp.int32, 16)
    vec = jnp.where(lane == 0, sel_smem[0],
                    jnp.where(lane == 1, sel_smem[1],
                              jnp.where(lane == 2, sel_smem[2],
                                        jnp.where(lane == 3, sel_smem[3],
                                                  _BIG))))
    obuf[...] = lax.sort(vec)
    pltpu.sync_copy(obuf, out_hbm.at[pl.ds(pl.multiple_of(wid * 16, 8), 16)])


_topk_sc = functools.partial(
    pl.kernel,
    out_type=jax.ShapeDtypeStruct((32 * 16,), jnp.int32),
    mesh=plsc.VectorSubcoreMesh(core_axis_name="c", subcore_axis_name="s",
                                num_cores=2, num_subcores=16),
    compiler_params=pltpu.CompilerParams(needs_layout_passes=False),
    scratch_types=[
        pltpu.VMEM((_D,), jnp.float32),
        pltpu.VMEM((_DC,), jnp.float32),
        pltpu.VMEM((16,), jnp.int32),
        pltpu.SMEM((_NSEG,), jnp.float32),
        pltpu.SMEM((8,), jnp.int32),
    ],
)(_topk_body)


def _onehot_body(idx_ref, out_ref):
    r = pl.program_id(0)
    s0 = idx_ref[r, 0]
    s1 = idx_ref[r, 1]
    s2 = idx_ref[r, 2]
    s3 = idx_ref[r, 3]
    riota = lax.broadcasted_iota(jnp.int32, (_K, 1), 0)
    srt = jnp.where(riota == 0, s0,
                    jnp.where(riota == 1, s1,
                              jnp.where(riota == 2, s2, s3)))
    col = lax.broadcasted_iota(jnp.int32, (_K, _D), 1)
    out_ref[0, 0] = (col == srt).astype(jnp.float32)


def kernel(inp, gn):
    n, d = inp.shape
    bs = gn.shape[0]
    idx = _topk_sc(inp.reshape(n * d), gn.reshape(bs * n * d)).reshape(bs * n, 16)
    out = pl.pallas_call(
        _onehot_body,
        grid=(bs * n,),
        in_specs=[pl.BlockSpec(memory_space=pltpu.SMEM)],
        out_specs=pl.BlockSpec((1, 1, _K, d), lambda r: (r // n, r % n, 0, 0)),
        out_shape=jax.ShapeDtypeStruct((bs, n, _K, d), jnp.float32),
    )(idx)
    return out
